# baseline (speedup 1.0000x reference)
import functools

import jax
import jax.numpy as jnp
from jax import lax
from jax.experimental import pallas as pl
from jax.experimental.pallas import tpu as pltpu
from jax.experimental.pallas import tpu_sc as plsc

N_VOCAB = 1000000
D_MODEL = 64
SEQLEN = 200
BATCH = 4096

NUM_WORKERS = 32
BTILE = 128
NBT = BATCH // BTILE
NT = (SEQLEN * NBT) // NUM_WORKERS
NB = 4
LANES = 16
S_PER_TILE = 7


def _make_body():
    mesh = plsc.VectorSubcoreMesh(core_axis_name="c", subcore_axis_name="s")

    @functools.partial(
        pl.kernel,
        mesh=mesh,
        compiler_params=pltpu.CompilerParams(
            use_tc_tiling_on_sc=False, needs_layout_passes=False),
        out_type=jax.ShapeDtypeStruct((SEQLEN, 8, NBT, 8, BTILE), jnp.float32),
        scratch_types=[
            pltpu.VMEM((NB, BTILE), jnp.int32),
            pltpu.VMEM((NB, BTILE, D_MODEL), jnp.float32),
            pltpu.VMEM((NB, D_MODEL, BTILE + 1), jnp.float32),
            pltpu.VMEM((SEQLEN, D_MODEL), jnp.float32),
            pltpu.VMEM_SHARED((SEQLEN // 2, D_MODEL, D_MODEL), jnp.float32),
            pltpu.SemaphoreType.DMA((NB,)),
            pltpu.SemaphoreType.DMA((NB,)),
            pltpu.SemaphoreType.DMA((NB,)),
            pltpu.SemaphoreType.DMA((NB,)),
        ],
    )
    def body(xt_hbm, wt_hbm, wp_hbm, out_hbm, idx_v, gath_v, trans_v, wp_v,
             post_sh, semg, sems, semp, semi):
        sid = lax.axis_index("s")
        cid = lax.axis_index("c")
        wid = cid * (NUM_WORKERS // 2) + sid
        t0 = wid * NT
        sbase = cid * (SEQLEN // 2)

        pltpu.sync_copy(wp_hbm, wp_v)

        lane_iota = lax.iota(jnp.int32, LANES)
        d_idx = [lane_iota + LANES * k for k in range(D_MODEL // LANES)]

        def fill_s(j, carry):
            s = sbase + sid * S_PER_TILE + j

            @pl.when(s < sbase + SEQLEN // 2)
            def _():
                s_vec = jnp.broadcast_to(s, (LANES,))

                def fill_d(d, carry2):
                    splat = plsc.load_gather(
                        wp_v, [s_vec, jnp.broadcast_to(d, (LANES,))])
                    for bb in range(BTILE // LANES):
                        trans_v[0, d, pl.ds(LANES * bb, LANES)] = splat
                    return carry2

                lax.fori_loop(0, D_MODEL, fill_d, 0, unroll=4)
                pltpu.sync_copy(trans_v.at[0, :, pl.ds(0, D_MODEL)],
                                post_sh.at[s - sbase])
            return carry

        lax.fori_loop(0, S_PER_TILE, fill_s, 0)
        plsc.subcore_barrier()

        def task_sb(t):
            return t // NBT, t % NBT

        def start(t, b):
            s, bt = task_sb(t)
            pltpu.make_async_copy(
                xt_hbm.at[s, pl.ds(bt * BTILE, BTILE)],
                idx_v.at[b], semi.at[b]).wait()
            pltpu.async_copy(
                wt_hbm.at[idx_v.at[b]], gath_v.at[b], semg.at[b])

            @pl.when(t - NB >= t0)
            def _():
                sp, btp = task_sb(t - NB)
                for db in range(8):
                    pltpu.make_async_copy(
                        trans_v.at[b, pl.ds(db * 8, 8), pl.ds(0, BTILE)],
                        out_hbm.at[sp, db, btp], sems.at[b]).wait()

            for h in range(2):
                pltpu.async_copy(
                    post_sh.at[s - sbase],
                    trans_v.at[b, :, pl.ds(h * D_MODEL, D_MODEL)], semp.at[b])

            @pl.when(t + 1 < t0 + NT)
            def _():
                s1, bt1 = task_sb(t + 1)
                pltpu.async_copy(
                    xt_hbm.at[s1, pl.ds(bt1 * BTILE, BTILE)],
                    idx_v.at[(b + 1) % NB], semi.at[(b + 1) % NB])

        def finish(t, b):
            s, bt = task_sb(t)
            pltpu.make_async_copy(
                wt_hbm.at[idx_v.at[b]], gath_v.at[b], semg.at[b]).wait()
            for h in range(2):
                pltpu.make_async_copy(
                    post_sh.at[s - sbase],
                    trans_v.at[b, :, pl.ds(h * D_MODEL, D_MODEL)],
                    semp.at[b]).wait()

            @plsc.parallel_loop(0, BTILE, unroll=16)
            def _(bi):
                b_vec = jnp.broadcast_to(bi, (LANES,))
                for k in range(D_MODEL // LANES):
                    v = gath_v[b, bi, pl.ds(LANES * k, LANES)]
                    plsc.addupdate_scatter(
                        trans_v.at[b], [d_idx[k], b_vec], v)

            for db in range(8):
                pltpu.async_copy(
                    trans_v.at[b, pl.ds(db * 8, 8), pl.ds(0, BTILE)],
                    out_hbm.at[s, db, bt], sems.at[b])

        pltpu.async_copy(
            xt_hbm.at[t0 // NBT, pl.ds((t0 % NBT) * BTILE, BTILE)],
            idx_v.at[0], semi.at[0])
        start(t0, 0)

        def step(i, carry):
            for b in range(NB):
                t = t0 + i * NB + b

                @pl.when(t > t0)
                def _():
                    start(t, b)
                    finish(t - 1, (b - 1) % NB)
            return carry

        lax.fori_loop(0, NT // NB, step, 0)

        finish(t0 + NT - 1, (NT - 1) % NB)
        for k in range(NB):
            t = t0 + NT - NB + k
            s, bt = task_sb(t)
            for db in range(8):
                pltpu.make_async_copy(
                    trans_v.at[t % NB, pl.ds(db * 8, 8), pl.ds(0, BTILE)],
                    out_hbm.at[s, db, bt], sems.at[t % NB]).wait()

    return body


_body = _make_body()


def kernel(x, W_tgt, W_pos):
    out5 = _body(x.T.astype(jnp.int32), W_tgt, W_pos)
    return out5.transpose(2, 4, 0, 1, 3).reshape(BATCH, SEQLEN, D_MODEL)


# speedup vs baseline: 1.0255x; 1.0255x over previous
import functools

import jax
import jax.numpy as jnp
from jax import lax
from jax.experimental import pallas as pl
from jax.experimental.pallas import tpu as pltpu
from jax.experimental.pallas import tpu_sc as plsc

N_VOCAB = 1000000
D_MODEL = 64
SEQLEN = 200
BATCH = 4096

NUM_WORKERS = 32
BTILE = 128
NBT = BATCH // BTILE
NT = (SEQLEN * NBT) // NUM_WORKERS
NB = 4
LANES = 16
S_PER_TILE = 7


def _make_body():
    mesh = plsc.VectorSubcoreMesh(core_axis_name="c", subcore_axis_name="s")

    @functools.partial(
        pl.kernel,
        mesh=mesh,
        compiler_params=pltpu.CompilerParams(
            use_tc_tiling_on_sc=False, needs_layout_passes=False),
        out_type=jax.ShapeDtypeStruct((SEQLEN, 8, NBT, 8, BTILE), jnp.float32),
        scratch_types=[
            pltpu.VMEM((NB, BTILE), jnp.int32),
            pltpu.VMEM((NB, BTILE, D_MODEL), jnp.float32),
            pltpu.VMEM((NB, D_MODEL, BTILE + 1), jnp.float32),
            pltpu.VMEM((SEQLEN, D_MODEL), jnp.float32),
            pltpu.VMEM_SHARED((SEQLEN // 2, D_MODEL, D_MODEL), jnp.float32),
            pltpu.SemaphoreType.DMA((NB,)),
            pltpu.SemaphoreType.DMA((NB,)),
            pltpu.SemaphoreType.DMA((NB,)),
            pltpu.SemaphoreType.DMA((NB,)),
        ],
    )
    def body(xt_hbm, wt_hbm, wp_hbm, out_hbm, idx_v, gath_v, trans_v, wp_v,
             post_sh, semg, sems, semp, semi):
        sid = lax.axis_index("s")
        cid = lax.axis_index("c")
        wid = cid * (NUM_WORKERS // 2) + sid
        t0 = wid * NT
        sbase = cid * (SEQLEN // 2)

        pltpu.sync_copy(wp_hbm, wp_v)

        lane_iota = lax.iota(jnp.int32, LANES)
        d_idx = [lane_iota + LANES * k for k in range(D_MODEL // LANES)]

        def fill_s(j, carry):
            s = sbase + sid * S_PER_TILE + j

            @pl.when(s < sbase + SEQLEN // 2)
            def _():
                s_vec = jnp.broadcast_to(s, (LANES,))

                def fill_d(d, carry2):
                    splat = plsc.load_gather(
                        wp_v, [s_vec, jnp.broadcast_to(d, (LANES,))])
                    for bb in range(BTILE // LANES):
                        trans_v[0, d, pl.ds(LANES * bb, LANES)] = splat
                    return carry2

                lax.fori_loop(0, D_MODEL, fill_d, 0, unroll=4)
                pltpu.sync_copy(trans_v.at[0, :, pl.ds(0, D_MODEL)],
                                post_sh.at[s - sbase])
            return carry

        lax.fori_loop(0, S_PER_TILE, fill_s, 0)
        plsc.subcore_barrier()

        def task_sb(t):
            return t // NBT, t % NBT

        def start(t, b):
            s, bt = task_sb(t)
            pltpu.make_async_copy(
                xt_hbm.at[s, pl.ds(bt * BTILE, BTILE)],
                idx_v.at[b], semi.at[b]).wait()
            pltpu.async_copy(
                wt_hbm.at[idx_v.at[b]], gath_v.at[b], semg.at[b])

            @pl.when(t - NB >= t0)
            def _():
                sp, btp = task_sb(t - NB)
                for db in range(8):
                    pltpu.make_async_copy(
                        trans_v.at[b, pl.ds(db * 8, 8), pl.ds(0, BTILE)],
                        out_hbm.at[sp, db, btp], sems.at[b]).wait()

            for h in range(2):
                pltpu.async_copy(
                    post_sh.at[s - sbase],
                    trans_v.at[b, :, pl.ds(h * D_MODEL, D_MODEL)], semp.at[b])

            @pl.when(t + 1 < t0 + NT)
            def _():
                s1, bt1 = task_sb(t + 1)
                pltpu.async_copy(
                    xt_hbm.at[s1, pl.ds(bt1 * BTILE, BTILE)],
                    idx_v.at[(b + 1) % NB], semi.at[(b + 1) % NB])

        def finish(t, b):
            s, bt = task_sb(t)
            pltpu.make_async_copy(
                wt_hbm.at[idx_v.at[b]], gath_v.at[b], semg.at[b]).wait()
            for h in range(2):
                pltpu.make_async_copy(
                    post_sh.at[s - sbase],
                    trans_v.at[b, :, pl.ds(h * D_MODEL, D_MODEL)],
                    semp.at[b]).wait()

            @plsc.parallel_loop(0, BTILE, unroll=8)
            def _(bi):
                b_vec = jnp.broadcast_to(bi, (LANES,))
                for k in range(D_MODEL // LANES):
                    v = gath_v[b, bi, pl.ds(LANES * k, LANES)]
                    plsc.addupdate_scatter(
                        trans_v.at[b], [d_idx[k], b_vec], v)

            for db in range(8):
                pltpu.async_copy(
                    trans_v.at[b, pl.ds(db * 8, 8), pl.ds(0, BTILE)],
                    out_hbm.at[s, db, bt], sems.at[b])

        pltpu.async_copy(
            xt_hbm.at[t0 // NBT, pl.ds((t0 % NBT) * BTILE, BTILE)],
            idx_v.at[0], semi.at[0])
        start(t0, 0)

        def step(i, carry):
            for b in range(NB):
                t = t0 + i * NB + b

                @pl.when(t > t0)
                def _():
                    start(t, b)
                    finish(t - 1, (b - 1) % NB)
            return carry

        lax.fori_loop(0, NT // NB, step, 0)

        finish(t0 + NT - 1, (NT - 1) % NB)
        for k in range(NB):
            t = t0 + NT - NB + k
            s, bt = task_sb(t)
            for db in range(8):
                pltpu.make_async_copy(
                    trans_v.at[t % NB, pl.ds(db * 8, 8), pl.ds(0, BTILE)],
                    out_hbm.at[s, db, bt], sems.at[t % NB]).wait()

    return body


_body = _make_body()


def kernel(x, W_tgt, W_pos):
    out5 = _body(x.T.astype(jnp.int32), W_tgt, W_pos)
    return out5.transpose(2, 4, 0, 1, 3).reshape(BATCH, SEQLEN, D_MODEL)


# speedup vs baseline: 1.0321x; 1.0064x over previous
import functools

import jax
import jax.numpy as jnp
from jax import lax
from jax.experimental import pallas as pl
from jax.experimental.pallas import tpu as pltpu
from jax.experimental.pallas import tpu_sc as plsc

N_VOCAB = 1000000
D_MODEL = 64
SEQLEN = 200
BATCH = 4096

NUM_WORKERS = 32
BTILE = 128
NBT = BATCH // BTILE
NT = (SEQLEN * NBT) // NUM_WORKERS
NB = 4
LANES = 16
S_PER_TILE = 7


def _make_body():
    mesh = plsc.VectorSubcoreMesh(core_axis_name="c", subcore_axis_name="s")

    @functools.partial(
        pl.kernel,
        mesh=mesh,
        compiler_params=pltpu.CompilerParams(
            use_tc_tiling_on_sc=False, needs_layout_passes=False),
        out_type=jax.ShapeDtypeStruct((SEQLEN, 8, NBT, 8, BTILE), jnp.float32),
        scratch_types=[
            pltpu.VMEM((NB, BTILE), jnp.int32),
            pltpu.VMEM((NB, BTILE, D_MODEL), jnp.float32),
            pltpu.VMEM((NB, D_MODEL, BTILE + 1), jnp.float32),
            pltpu.VMEM((SEQLEN, D_MODEL), jnp.float32),
            pltpu.VMEM_SHARED((SEQLEN // 2, D_MODEL, D_MODEL), jnp.float32),
            pltpu.SemaphoreType.DMA((NB,)),
            pltpu.SemaphoreType.DMA((NB,)),
            pltpu.SemaphoreType.DMA((NB,)),
            pltpu.SemaphoreType.DMA((NB,)),
        ],
    )
    def body(xt_hbm, wt_hbm, wp_hbm, out_hbm, idx_v, gath_v, trans_v, wp_v,
             post_sh, semg, sems, semp, semi):
        sid = lax.axis_index("s")
        cid = lax.axis_index("c")
        wid = cid * (NUM_WORKERS // 2) + sid
        t0 = wid * NT
        sbase = cid * (SEQLEN // 2)

        pltpu.sync_copy(wp_hbm, wp_v)

        lane_iota = lax.iota(jnp.int32, LANES)
        d_idx = [lane_iota + LANES * k for k in range(D_MODEL // LANES)]

        def fill_s(j, carry):
            s = sbase + sid * S_PER_TILE + j

            @pl.when(s < sbase + SEQLEN // 2)
            def _():
                s_vec = jnp.broadcast_to(s, (LANES,))

                def fill_d(d, carry2):
                    splat = plsc.load_gather(
                        wp_v, [s_vec, jnp.broadcast_to(d, (LANES,))])
                    for bb in range(BTILE // LANES):
                        trans_v[0, d, pl.ds(LANES * bb, LANES)] = splat
                    return carry2

                lax.fori_loop(0, D_MODEL, fill_d, 0, unroll=4)
                pltpu.sync_copy(trans_v.at[0, :, pl.ds(0, D_MODEL)],
                                post_sh.at[s - sbase])
            return carry

        lax.fori_loop(0, S_PER_TILE, fill_s, 0)
        plsc.subcore_barrier()

        def task_sb(t):
            return t // NBT, t % NBT

        def start(t, b):
            s, bt = task_sb(t)
            pltpu.make_async_copy(
                xt_hbm.at[s, pl.ds(bt * BTILE, BTILE)],
                idx_v.at[b], semi.at[b]).wait()
            pltpu.async_copy(
                wt_hbm.at[idx_v.at[b]], gath_v.at[b], semg.at[b])

            @pl.when(t - NB >= t0)
            def _():
                sp, btp = task_sb(t - NB)
                for db in range(8):
                    pltpu.make_async_copy(
                        trans_v.at[b, pl.ds(db * 8, 8), pl.ds(0, BTILE)],
                        out_hbm.at[sp, db, btp], sems.at[b]).wait()

            for h in range(2):
                pltpu.async_copy(
                    post_sh.at[s - sbase],
                    trans_v.at[b, :, pl.ds(h * D_MODEL, D_MODEL)], semp.at[b])

            @pl.when(t + 1 < t0 + NT)
            def _():
                s1, bt1 = task_sb(t + 1)
                pltpu.async_copy(
                    xt_hbm.at[s1, pl.ds(bt1 * BTILE, BTILE)],
                    idx_v.at[(b + 1) % NB], semi.at[(b + 1) % NB])

        def finish(t, b):
            s, bt = task_sb(t)
            pltpu.make_async_copy(
                wt_hbm.at[idx_v.at[b]], gath_v.at[b], semg.at[b]).wait()
            for h in range(2):
                pltpu.make_async_copy(
                    post_sh.at[s - sbase],
                    trans_v.at[b, :, pl.ds(h * D_MODEL, D_MODEL)],
                    semp.at[b]).wait()

            @plsc.parallel_loop(0, BTILE, unroll=4)
            def _(bi):
                b_vec = jnp.broadcast_to(bi, (LANES,))
                for k in range(D_MODEL // LANES):
                    v = gath_v[b, bi, pl.ds(LANES * k, LANES)]
                    plsc.addupdate_scatter(
                        trans_v.at[b], [d_idx[k], b_vec], v)

            for db in range(8):
                pltpu.async_copy(
                    trans_v.at[b, pl.ds(db * 8, 8), pl.ds(0, BTILE)],
                    out_hbm.at[s, db, bt], sems.at[b])

        pltpu.async_copy(
            xt_hbm.at[t0 // NBT, pl.ds((t0 % NBT) * BTILE, BTILE)],
            idx_v.at[0], semi.at[0])
        start(t0, 0)

        def step(i, carry):
            for b in range(NB):
                t = t0 + i * NB + b

                @pl.when(t > t0)
                def _():
                    start(t, b)
                    finish(t - 1, (b - 1) % NB)
            return carry

        lax.fori_loop(0, NT // NB, step, 0)

        finish(t0 + NT - 1, (NT - 1) % NB)
        for k in range(NB):
            t = t0 + NT - NB + k
            s, bt = task_sb(t)
            for db in range(8):
                pltpu.make_async_copy(
                    trans_v.at[t % NB, pl.ds(db * 8, 8), pl.ds(0, BTILE)],
                    out_hbm.at[s, db, bt], sems.at[t % NB]).wait()

    return body


_body = _make_body()


def kernel(x, W_tgt, W_pos):
    out5 = _body(x.T.astype(jnp.int32), W_tgt, W_pos)
    return out5.transpose(2, 4, 0, 1, 3).reshape(BATCH, SEQLEN, D_MODEL)
